# Initial kernel scaffold; baseline (speedup 1.0000x reference)
import jax
import jax.numpy as jnp
from jax.experimental import pallas as pl


def kernel(x, edge_index, basis_lin_msg_wt, basis_lin_msg_biases, basis_lin_self_wt, basis_lin_self_biases, linear_combinations):
    raise NotImplementedError("write your pallas kernel here")


# speedup vs baseline: 9.9762x; 9.9762x over previous
import functools

import jax
import jax.numpy as jnp
from jax import lax
from jax.experimental import pallas as pl
from jax.experimental.pallas import tpu as pltpu
from jax.experimental.pallas import tpu_sc as plsc

N_NODES = 10000
N_EDGES = 320000
D = 128
DH = D // 2
NB = 4
NC = 2
NS = 16
EPW = N_EDGES // NS
C = 80
NCH = EPW // C
NBUF = 5
GRP = NCH // NBUF
NPAD = 10240
RPS = NPAD // NS
ZCH = 80
DEGW = 16


def _sc_body(x0_hbm, x1_hbm, src_hbm, dst_hbm, zacc_hbm, zdeg_hbm, ones_hbm,
             acc_out, deg_out,
             acc_sh, deg_sh, sidx_v, didx_v,
             r0_v, r1_v, r2_v, r3_v, r4_v, w16_v,
             g0, g1, g2, g3, g4, s0, s1, s2, s3, s4, dsem):
    c = lax.axis_index("c")
    s = lax.axis_index("s")
    r0 = s * RPS
    bufs = (r0_v, r1_v, r2_v, r3_v, r4_v)
    gsems = (g0, g1, g2, g3, g4)
    ssems = (s0, s1, s2, s3, s4)

    is0 = c == 0
    def deg_on(i):
        del i
        return is0

    pltpu.sync_copy(zacc_hbm, r0_v)
    for j in range(RPS // ZCH):
        pltpu.sync_copy(r0_v, acc_sh.at[pl.ds(r0 + j * ZCH, ZCH)])

    @pl.when(is0)
    def _():
        pltpu.sync_copy(zdeg_hbm, w16_v)
        for j in range(RPS // ZCH):
            pltpu.sync_copy(w16_v, deg_sh.at[pl.ds(r0 + j * ZCH, ZCH)])

    pltpu.sync_copy(ones_hbm, w16_v)
    pltpu.sync_copy(src_hbm.at[s], sidx_v)
    pltpu.sync_copy(dst_hbm.at[s], didx_v)
    plsc.subcore_barrier()

    def gather(k, buf, sem):
        @pl.when(is0)
        def _():
            pltpu.async_copy(x0_hbm.at[sidx_v.at[k]], buf, sem)

        @pl.when(jnp.logical_not(is0))
        def _():
            pltpu.async_copy(x1_hbm.at[sidx_v.at[k]], buf, sem)

    def wait_gather(buf, sem):
        pltpu.make_async_copy(x0_hbm.at[sidx_v.at[0]], buf, sem).wait()

    for b in range(NBUF):
        gather(b, bufs[b], gsems[b])

    def group(i, carry):
        k0 = NBUF * i
        on = deg_on(i)
        for b in range(NBUF):
            k = k0 + b
            wait_gather(bufs[b], gsems[b])
            pltpu.async_copy(bufs[b], acc_sh.at[didx_v.at[k]], ssems[b],
                             add=True)

            @pl.when(on)
            def _():
                pltpu.async_copy(w16_v, deg_sh.at[didx_v.at[k]], dsem,
                                 add=True)

        for b in range(NBUF):
            pltpu.make_async_copy(bufs[b], acc_sh.at[didx_v.at[0]],
                                  ssems[b]).wait()

            @pl.when(i + 1 < GRP)
            def _():
                gather(k0 + NBUF + b, bufs[b], gsems[b])

        for b in range(NBUF):
            @pl.when(on)
            def _():
                pltpu.make_async_copy(w16_v, deg_sh.at[didx_v.at[0]],
                                      dsem).wait()
        return carry

    lax.fori_loop(0, GRP, group, 0)
    plsc.subcore_barrier()

    for j in range(RPS // ZCH):
        pltpu.sync_copy(acc_sh.at[pl.ds(r0 + j * ZCH, ZCH)], r0_v)
        pltpu.sync_copy(r0_v, acc_out.at[c, pl.ds(r0 + j * ZCH, ZCH)])

    @pl.when(is0)
    def _():
        for j in range(RPS // ZCH):
            pltpu.sync_copy(deg_sh.at[pl.ds(r0 + j * ZCH, ZCH)], w16_v)
            pltpu.sync_copy(w16_v, deg_out.at[pl.ds(r0 + j * ZCH, ZCH)])


_sc_scatter = functools.partial(
    pl.kernel,
    out_type=(jax.ShapeDtypeStruct((NC, NPAD, DH), jnp.float32),
              jax.ShapeDtypeStruct((NPAD, DEGW), jnp.float32)),
    mesh=plsc.VectorSubcoreMesh(core_axis_name="c", subcore_axis_name="s"),
    compiler_params=pltpu.CompilerParams(use_tc_tiling_on_sc=False),
    scratch_types=[
        pltpu.VMEM_SHARED((NPAD, DH), jnp.float32),
        pltpu.VMEM_SHARED((NPAD, DEGW), jnp.float32),
        pltpu.VMEM((NCH, C), jnp.int32),
        pltpu.VMEM((NCH, C), jnp.int32),
        pltpu.VMEM((C, DH), jnp.float32),
        pltpu.VMEM((C, DH), jnp.float32),
        pltpu.VMEM((C, DH), jnp.float32),
        pltpu.VMEM((C, DH), jnp.float32),
        pltpu.VMEM((C, DH), jnp.float32),
        pltpu.VMEM((ZCH, DEGW), jnp.float32),
        pltpu.SemaphoreType.DMA,
        pltpu.SemaphoreType.DMA,
        pltpu.SemaphoreType.DMA,
        pltpu.SemaphoreType.DMA,
        pltpu.SemaphoreType.DMA,
        pltpu.SemaphoreType.DMA,
        pltpu.SemaphoreType.DMA,
        pltpu.SemaphoreType.DMA,
        pltpu.SemaphoreType.DMA,
        pltpu.SemaphoreType.DMA,
        pltpu.SemaphoreType.DMA,
    ],
)(_sc_body)


RB = 2000


def _tc_body(acc_ref, deg_ref, x_ref, bmw_ref, bmb_ref, bsw_ref, bsb_ref,
             coef_ref, out_ref):
    c0 = coef_ref[0, 0]
    c1 = coef_ref[0, 1]
    c2 = coef_ref[0, 2]
    c3 = coef_ref[0, 3]
    wm = c0 * bmw_ref[0] + c1 * bmw_ref[1] + c2 * bmw_ref[2] + c3 * bmw_ref[3]
    ws = c0 * bsw_ref[0] + c1 * bsw_ref[1] + c2 * bsw_ref[2] + c3 * bsw_ref[3]
    bm = c0 * bmb_ref[0] + c1 * bmb_ref[1] + c2 * bmb_ref[2] + c3 * bmb_ref[3]
    bs = c0 * bsb_ref[0] + c1 * bsb_ref[1] + c2 * bsb_ref[2] + c3 * bsb_ref[3]

    deg = deg_ref[:, 0:1]
    out = (jnp.dot(acc_ref[0], wm[:DH], preferred_element_type=jnp.float32,
                   precision=lax.Precision.HIGHEST)
           + jnp.dot(acc_ref[1], wm[DH:], preferred_element_type=jnp.float32,
                     precision=lax.Precision.HIGHEST)
           + jnp.dot(x_ref[...], ws, preferred_element_type=jnp.float32,
                     precision=lax.Precision.HIGHEST)
           + deg * bm.reshape(1, D) + bs.reshape(1, D))
    nrm = jnp.sqrt(jnp.sum(out * out, axis=1, keepdims=True))
    out_ref[...] = out / jnp.maximum(nrm, 1e-12)


def _tc_combine(acc, deg, x, bmw, bmb, bsw, bsb, coef):
    return pl.pallas_call(
        _tc_body,
        grid=(N_NODES // RB,),
        in_specs=[
            pl.BlockSpec((NC, RB, DH), lambda i: (0, i, 0)),
            pl.BlockSpec((RB, DEGW), lambda i: (i, 0)),
            pl.BlockSpec((RB, D), lambda i: (i, 0)),
            pl.BlockSpec((NB, D, D), lambda i: (0, 0, 0)),
            pl.BlockSpec((NB, D), lambda i: (0, 0)),
            pl.BlockSpec((NB, D, D), lambda i: (0, 0, 0)),
            pl.BlockSpec((NB, D), lambda i: (0, 0)),
            pl.BlockSpec((1, NB), lambda i: (0, 0)),
        ],
        out_specs=pl.BlockSpec((RB, D), lambda i: (i, 0)),
        out_shape=jax.ShapeDtypeStruct((N_NODES, D), jnp.float32),
    )(acc, deg, x, bmw, bmb, bsw, bsb, coef)


def kernel(x, edge_index, basis_lin_msg_wt, basis_lin_msg_biases,
           basis_lin_self_wt, basis_lin_self_biases, linear_combinations):
    src = edge_index[0].astype(jnp.int32).reshape(NS, NCH, C)
    dst = edge_index[1].astype(jnp.int32).reshape(NS, NCH, C)
    x0 = x[:, :DH]
    x1 = x[:, DH:]
    zacc = jnp.zeros((ZCH, DH), jnp.float32)
    zdeg = jnp.zeros((ZCH, DEGW), jnp.float32)
    ones = jnp.ones((C, DEGW), jnp.float32)

    acc, deg = _sc_scatter(x0, x1, src, dst, zacc, zdeg, ones)

    bmw = jnp.transpose(basis_lin_msg_wt, (2, 0, 1))
    bsw = jnp.transpose(basis_lin_self_wt, (2, 0, 1))
    bmb = jnp.transpose(basis_lin_msg_biases, (1, 0))
    bsb = jnp.transpose(basis_lin_self_biases, (1, 0))
    return _tc_combine(acc, deg, x, bmw, bmb, bsw, bsb, linear_combinations)


# speedup vs baseline: 10.0379x; 1.0062x over previous
import functools

import jax
import jax.numpy as jnp
from jax import lax
from jax.experimental import pallas as pl
from jax.experimental.pallas import tpu as pltpu
from jax.experimental.pallas import tpu_sc as plsc

N_NODES = 10000
N_EDGES = 320000
D = 128
DH = D // 2
NB = 4
NC = 2
NS = 16
EPW = N_EDGES // NS
C = 80
NCH = EPW // C
NBUF = 5
GRP = NCH // NBUF
NPAD = 10240
RPS = NPAD // NS
ZCH = 80
DEGW = 16


def _sc_body(x0_hbm, x1_hbm, src_hbm, dst_hbm, zacc_hbm, zdeg_hbm, ones_hbm,
             acc_out, deg_out,
             acc_sh, deg_sh, sidx_v, didx_v,
             r0_v, r1_v, r2_v, r3_v, r4_v, w16_v,
             g0, g1, g2, g3, g4, s0, s1, s2, s3, s4, dsem):
    c = lax.axis_index("c")
    s = lax.axis_index("s")
    r0 = s * RPS
    bufs = (r0_v, r1_v, r2_v, r3_v, r4_v)
    gsems = (g0, g1, g2, g3, g4)
    ssems = (s0, s1, s2, s3, s4)

    is0 = c == 0
    def deg_on(i):
        return (i < GRP // 2) == is0

    pltpu.sync_copy(zacc_hbm, r0_v)
    for j in range(RPS // ZCH):
        pltpu.sync_copy(r0_v, acc_sh.at[pl.ds(r0 + j * ZCH, ZCH)])

    pltpu.sync_copy(zdeg_hbm, w16_v)
    for j in range(RPS // ZCH):
        pltpu.sync_copy(w16_v, deg_sh.at[pl.ds(r0 + j * ZCH, ZCH)])

    pltpu.sync_copy(ones_hbm, w16_v)
    pltpu.sync_copy(src_hbm.at[s], sidx_v)
    pltpu.sync_copy(dst_hbm.at[s], didx_v)
    plsc.subcore_barrier()

    def gather(k, buf, sem):
        @pl.when(is0)
        def _():
            pltpu.async_copy(x0_hbm.at[sidx_v.at[k]], buf, sem)

        @pl.when(jnp.logical_not(is0))
        def _():
            pltpu.async_copy(x1_hbm.at[sidx_v.at[k]], buf, sem)

    def wait_gather(buf, sem):
        pltpu.make_async_copy(x0_hbm.at[sidx_v.at[0]], buf, sem).wait()

    for b in range(NBUF):
        gather(b, bufs[b], gsems[b])

    def group(i, carry):
        k0 = NBUF * i
        on = deg_on(i)
        for b in range(NBUF):
            k = k0 + b
            wait_gather(bufs[b], gsems[b])
            pltpu.async_copy(bufs[b], acc_sh.at[didx_v.at[k]], ssems[b],
                             add=True)

            @pl.when(on)
            def _():
                pltpu.async_copy(w16_v, deg_sh.at[didx_v.at[k]], dsem,
                                 add=True)

        for b in range(NBUF):
            pltpu.make_async_copy(bufs[b], acc_sh.at[didx_v.at[0]],
                                  ssems[b]).wait()

            @pl.when(i + 1 < GRP)
            def _():
                gather(k0 + NBUF + b, bufs[b], gsems[b])

        for b in range(NBUF):
            @pl.when(on)
            def _():
                pltpu.make_async_copy(w16_v, deg_sh.at[didx_v.at[0]],
                                      dsem).wait()
        return carry

    lax.fori_loop(0, GRP, group, 0)
    plsc.subcore_barrier()

    for j in range(RPS // ZCH):
        pltpu.sync_copy(acc_sh.at[pl.ds(r0 + j * ZCH, ZCH)], r0_v)
        pltpu.sync_copy(r0_v, acc_out.at[c, pl.ds(r0 + j * ZCH, ZCH)])

    for j in range(RPS // ZCH):
        pltpu.sync_copy(deg_sh.at[pl.ds(r0 + j * ZCH, ZCH)], w16_v)
        pltpu.sync_copy(w16_v, deg_out.at[c, pl.ds(r0 + j * ZCH, ZCH)])


_sc_scatter = functools.partial(
    pl.kernel,
    out_type=(jax.ShapeDtypeStruct((NC, NPAD, DH), jnp.float32),
              jax.ShapeDtypeStruct((NC, NPAD, DEGW), jnp.float32)),
    mesh=plsc.VectorSubcoreMesh(core_axis_name="c", subcore_axis_name="s"),
    compiler_params=pltpu.CompilerParams(use_tc_tiling_on_sc=False),
    scratch_types=[
        pltpu.VMEM_SHARED((NPAD, DH), jnp.float32),
        pltpu.VMEM_SHARED((NPAD, DEGW), jnp.float32),
        pltpu.VMEM((NCH, C), jnp.int32),
        pltpu.VMEM((NCH, C), jnp.int32),
        pltpu.VMEM((C, DH), jnp.float32),
        pltpu.VMEM((C, DH), jnp.float32),
        pltpu.VMEM((C, DH), jnp.float32),
        pltpu.VMEM((C, DH), jnp.float32),
        pltpu.VMEM((C, DH), jnp.float32),
        pltpu.VMEM((ZCH, DEGW), jnp.float32),
        pltpu.SemaphoreType.DMA,
        pltpu.SemaphoreType.DMA,
        pltpu.SemaphoreType.DMA,
        pltpu.SemaphoreType.DMA,
        pltpu.SemaphoreType.DMA,
        pltpu.SemaphoreType.DMA,
        pltpu.SemaphoreType.DMA,
        pltpu.SemaphoreType.DMA,
        pltpu.SemaphoreType.DMA,
        pltpu.SemaphoreType.DMA,
        pltpu.SemaphoreType.DMA,
    ],
)(_sc_body)


RB = 2000


def _tc_body(acc_ref, deg_ref, x_ref, bmw_ref, bmb_ref, bsw_ref, bsb_ref,
             coef_ref, out_ref):
    c0 = coef_ref[0, 0]
    c1 = coef_ref[0, 1]
    c2 = coef_ref[0, 2]
    c3 = coef_ref[0, 3]
    wm = c0 * bmw_ref[0] + c1 * bmw_ref[1] + c2 * bmw_ref[2] + c3 * bmw_ref[3]
    ws = c0 * bsw_ref[0] + c1 * bsw_ref[1] + c2 * bsw_ref[2] + c3 * bsw_ref[3]
    bm = c0 * bmb_ref[0] + c1 * bmb_ref[1] + c2 * bmb_ref[2] + c3 * bmb_ref[3]
    bs = c0 * bsb_ref[0] + c1 * bsb_ref[1] + c2 * bsb_ref[2] + c3 * bsb_ref[3]

    deg = deg_ref[0, :, 0:1] + deg_ref[1, :, 0:1]
    out = (jnp.dot(acc_ref[0], wm[:DH], preferred_element_type=jnp.float32,
                   precision=lax.Precision.HIGHEST)
           + jnp.dot(acc_ref[1], wm[DH:], preferred_element_type=jnp.float32,
                     precision=lax.Precision.HIGHEST)
           + jnp.dot(x_ref[...], ws, preferred_element_type=jnp.float32,
                     precision=lax.Precision.HIGHEST)
           + deg * bm.reshape(1, D) + bs.reshape(1, D))
    nrm = jnp.sqrt(jnp.sum(out * out, axis=1, keepdims=True))
    out_ref[...] = out / jnp.maximum(nrm, 1e-12)


def _tc_combine(acc, deg, x, bmw, bmb, bsw, bsb, coef):
    return pl.pallas_call(
        _tc_body,
        grid=(N_NODES // RB,),
        in_specs=[
            pl.BlockSpec((NC, RB, DH), lambda i: (0, i, 0)),
            pl.BlockSpec((NC, RB, DEGW), lambda i: (0, i, 0)),
            pl.BlockSpec((RB, D), lambda i: (i, 0)),
            pl.BlockSpec((NB, D, D), lambda i: (0, 0, 0)),
            pl.BlockSpec((NB, D), lambda i: (0, 0)),
            pl.BlockSpec((NB, D, D), lambda i: (0, 0, 0)),
            pl.BlockSpec((NB, D), lambda i: (0, 0)),
            pl.BlockSpec((1, NB), lambda i: (0, 0)),
        ],
        out_specs=pl.BlockSpec((RB, D), lambda i: (i, 0)),
        out_shape=jax.ShapeDtypeStruct((N_NODES, D), jnp.float32),
    )(acc, deg, x, bmw, bmb, bsw, bsb, coef)


def kernel(x, edge_index, basis_lin_msg_wt, basis_lin_msg_biases,
           basis_lin_self_wt, basis_lin_self_biases, linear_combinations):
    src = edge_index[0].astype(jnp.int32).reshape(NS, NCH, C)
    dst = edge_index[1].astype(jnp.int32).reshape(NS, NCH, C)
    x0 = x[:, :DH]
    x1 = x[:, DH:]
    zacc = jnp.zeros((ZCH, DH), jnp.float32)
    zdeg = jnp.zeros((ZCH, DEGW), jnp.float32)
    ones = jnp.ones((C, DEGW), jnp.float32)

    acc, deg = _sc_scatter(x0, x1, src, dst, zacc, zdeg, ones)

    bmw = jnp.transpose(basis_lin_msg_wt, (2, 0, 1))
    bsw = jnp.transpose(basis_lin_self_wt, (2, 0, 1))
    bmb = jnp.transpose(basis_lin_msg_biases, (1, 0))
    bsb = jnp.transpose(basis_lin_self_biases, (1, 0))
    return _tc_combine(acc, deg, x, bmw, bmb, bsw, bsb, linear_combinations)


# speedup vs baseline: 10.3429x; 1.0304x over previous
import functools

import jax
import jax.numpy as jnp
from jax import lax
from jax.experimental import pallas as pl
from jax.experimental.pallas import tpu as pltpu
from jax.experimental.pallas import tpu_sc as plsc

N_NODES = 10000
N_EDGES = 320000
D = 128
DH = D // 2
NB = 4
NC = 2
NS = 16
EPW = N_EDGES // NS
C = 80
NCH = EPW // C
NBUF = 5
GRP = NCH // NBUF
NPAD = 10240
RPS = NPAD // NS
ZCH = 80
DEGW = 16


def _sc_body(x0_hbm, x1_hbm, src_hbm, dst_hbm, zacc_hbm, zdeg_hbm, ones_hbm,
             acc_out, deg_out,
             acc_sh, deg_sh, sidx_v, didx_v,
             r0_v, r1_v, r2_v, r3_v, r4_v, w16_v,
             g0, g1, g2, g3, g4, s0, s1, s2, s3, s4, dsem):
    c = lax.axis_index("c")
    s = lax.axis_index("s")
    r0 = s * RPS
    bufs = (r0_v, r1_v, r2_v, r3_v, r4_v)
    gsems = (g0, g1, g2, g3, g4)
    ssems = (s0, s1, s2, s3, s4)

    is0 = c == 0
    def deg_on(i):
        return (i < GRP // 2) == is0

    pltpu.async_copy(src_hbm.at[s], sidx_v, s3)
    pltpu.async_copy(dst_hbm.at[s], didx_v, s4)

    pltpu.sync_copy(zacc_hbm, r0_v)
    for j in range(RPS // ZCH):
        pltpu.async_copy(r0_v, acc_sh.at[pl.ds(r0 + j * ZCH, ZCH)],
                         gsems[j % 3])
    pltpu.sync_copy(zdeg_hbm, w16_v)
    for j in range(RPS // ZCH):
        pltpu.async_copy(w16_v, deg_sh.at[pl.ds(r0 + j * ZCH, ZCH)], dsem)
    for j in range(RPS // ZCH):
        pltpu.make_async_copy(r0_v, acc_sh.at[pl.ds(r0 + j * ZCH, ZCH)],
                              gsems[j % 3]).wait()
        pltpu.make_async_copy(w16_v, deg_sh.at[pl.ds(r0 + j * ZCH, ZCH)],
                              dsem).wait()
    pltpu.sync_copy(ones_hbm, w16_v)
    pltpu.make_async_copy(src_hbm.at[s], sidx_v, s3).wait()
    pltpu.make_async_copy(dst_hbm.at[s], didx_v, s4).wait()
    plsc.subcore_barrier()

    def gather(k, buf, sem):
        @pl.when(is0)
        def _():
            pltpu.async_copy(x0_hbm.at[sidx_v.at[k]], buf, sem)

        @pl.when(jnp.logical_not(is0))
        def _():
            pltpu.async_copy(x1_hbm.at[sidx_v.at[k]], buf, sem)

    def wait_gather(buf, sem):
        pltpu.make_async_copy(x0_hbm.at[sidx_v.at[0]], buf, sem).wait()

    for b in range(NBUF):
        gather(b, bufs[b], gsems[b])

    def group(i, carry):
        k0 = NBUF * i
        on = deg_on(i)
        for b in range(NBUF):
            k = k0 + b
            wait_gather(bufs[b], gsems[b])
            pltpu.async_copy(bufs[b], acc_sh.at[didx_v.at[k]], ssems[b],
                             add=True)

            @pl.when(on)
            def _():
                pltpu.async_copy(w16_v, deg_sh.at[didx_v.at[k]], dsem,
                                 add=True)

        for b in range(NBUF):
            pltpu.make_async_copy(bufs[b], acc_sh.at[didx_v.at[0]],
                                  ssems[b]).wait()

            @pl.when(i + 1 < GRP)
            def _():
                gather(k0 + NBUF + b, bufs[b], gsems[b])

        for b in range(NBUF):
            @pl.when(on)
            def _():
                pltpu.make_async_copy(w16_v, deg_sh.at[didx_v.at[0]],
                                      dsem).wait()
        return carry

    lax.fori_loop(0, GRP, group, 0)
    plsc.subcore_barrier()

    for j in range(RPS // ZCH):
        b = j % NBUF
        if j >= NBUF:
            pltpu.make_async_copy(
                bufs[b], acc_out.at[c, pl.ds(r0 + (j - NBUF) * ZCH, ZCH)],
                ssems[b]).wait()
        pltpu.sync_copy(acc_sh.at[pl.ds(r0 + j * ZCH, ZCH)], bufs[b])
        pltpu.async_copy(bufs[b], acc_out.at[c, pl.ds(r0 + j * ZCH, ZCH)],
                         ssems[b])

    for j in range(RPS // ZCH):
        pltpu.sync_copy(deg_sh.at[pl.ds(r0 + j * ZCH, ZCH)], w16_v)
        pltpu.sync_copy(w16_v, deg_out.at[c, pl.ds(r0 + j * ZCH, ZCH)])

    for j in range(RPS // ZCH - NBUF, RPS // ZCH):
        b = j % NBUF
        pltpu.make_async_copy(bufs[b],
                              acc_out.at[c, pl.ds(r0 + j * ZCH, ZCH)],
                              ssems[b]).wait()


_sc_scatter = functools.partial(
    pl.kernel,
    out_type=(jax.ShapeDtypeStruct((NC, NPAD, DH), jnp.float32),
              jax.ShapeDtypeStruct((NC, NPAD, DEGW), jnp.float32)),
    mesh=plsc.VectorSubcoreMesh(core_axis_name="c", subcore_axis_name="s"),
    compiler_params=pltpu.CompilerParams(use_tc_tiling_on_sc=False),
    scratch_types=[
        pltpu.VMEM_SHARED((NPAD, DH), jnp.float32),
        pltpu.VMEM_SHARED((NPAD, DEGW), jnp.float32),
        pltpu.VMEM((NCH, C), jnp.int32),
        pltpu.VMEM((NCH, C), jnp.int32),
        pltpu.VMEM((C, DH), jnp.float32),
        pltpu.VMEM((C, DH), jnp.float32),
        pltpu.VMEM((C, DH), jnp.float32),
        pltpu.VMEM((C, DH), jnp.float32),
        pltpu.VMEM((C, DH), jnp.float32),
        pltpu.VMEM((ZCH, DEGW), jnp.float32),
        pltpu.SemaphoreType.DMA,
        pltpu.SemaphoreType.DMA,
        pltpu.SemaphoreType.DMA,
        pltpu.SemaphoreType.DMA,
        pltpu.SemaphoreType.DMA,
        pltpu.SemaphoreType.DMA,
        pltpu.SemaphoreType.DMA,
        pltpu.SemaphoreType.DMA,
        pltpu.SemaphoreType.DMA,
        pltpu.SemaphoreType.DMA,
        pltpu.SemaphoreType.DMA,
    ],
)(_sc_body)


RB = 2000


def _tc_body(acc_ref, deg_ref, x_ref, bmw_ref, bmb_ref, bsw_ref, bsb_ref,
             coef_ref, out_ref):
    c0 = coef_ref[0, 0]
    c1 = coef_ref[0, 1]
    c2 = coef_ref[0, 2]
    c3 = coef_ref[0, 3]
    wm = c0 * bmw_ref[0] + c1 * bmw_ref[1] + c2 * bmw_ref[2] + c3 * bmw_ref[3]
    ws = c0 * bsw_ref[0] + c1 * bsw_ref[1] + c2 * bsw_ref[2] + c3 * bsw_ref[3]
    bm = c0 * bmb_ref[0] + c1 * bmb_ref[1] + c2 * bmb_ref[2] + c3 * bmb_ref[3]
    bs = c0 * bsb_ref[0] + c1 * bsb_ref[1] + c2 * bsb_ref[2] + c3 * bsb_ref[3]

    deg = deg_ref[0, :, 0:1] + deg_ref[1, :, 0:1]
    out = (jnp.dot(acc_ref[0], wm[:DH], preferred_element_type=jnp.float32,
                   precision=lax.Precision.HIGHEST)
           + jnp.dot(acc_ref[1], wm[DH:], preferred_element_type=jnp.float32,
                     precision=lax.Precision.HIGHEST)
           + jnp.dot(x_ref[...], ws, preferred_element_type=jnp.float32,
                     precision=lax.Precision.HIGHEST)
           + deg * bm.reshape(1, D) + bs.reshape(1, D))
    nrm = jnp.sqrt(jnp.sum(out * out, axis=1, keepdims=True))
    out_ref[...] = out / jnp.maximum(nrm, 1e-12)


def _tc_combine(acc, deg, x, bmw, bmb, bsw, bsb, coef):
    return pl.pallas_call(
        _tc_body,
        grid=(N_NODES // RB,),
        in_specs=[
            pl.BlockSpec((NC, RB, DH), lambda i: (0, i, 0)),
            pl.BlockSpec((NC, RB, DEGW), lambda i: (0, i, 0)),
            pl.BlockSpec((RB, D), lambda i: (i, 0)),
            pl.BlockSpec((NB, D, D), lambda i: (0, 0, 0)),
            pl.BlockSpec((NB, D), lambda i: (0, 0)),
            pl.BlockSpec((NB, D, D), lambda i: (0, 0, 0)),
            pl.BlockSpec((NB, D), lambda i: (0, 0)),
            pl.BlockSpec((1, NB), lambda i: (0, 0)),
        ],
        out_specs=pl.BlockSpec((RB, D), lambda i: (i, 0)),
        out_shape=jax.ShapeDtypeStruct((N_NODES, D), jnp.float32),
    )(acc, deg, x, bmw, bmb, bsw, bsb, coef)


def kernel(x, edge_index, basis_lin_msg_wt, basis_lin_msg_biases,
           basis_lin_self_wt, basis_lin_self_biases, linear_combinations):
    src = edge_index[0].astype(jnp.int32).reshape(NS, NCH, C)
    dst = edge_index[1].astype(jnp.int32).reshape(NS, NCH, C)
    x0 = x[:, :DH]
    x1 = x[:, DH:]
    zacc = jnp.zeros((ZCH, DH), jnp.float32)
    zdeg = jnp.zeros((ZCH, DEGW), jnp.float32)
    ones = jnp.ones((C, DEGW), jnp.float32)

    acc, deg = _sc_scatter(x0, x1, src, dst, zacc, zdeg, ones)

    bmw = jnp.transpose(basis_lin_msg_wt, (2, 0, 1))
    bsw = jnp.transpose(basis_lin_self_wt, (2, 0, 1))
    bmb = jnp.transpose(basis_lin_msg_biases, (1, 0))
    bsb = jnp.transpose(basis_lin_self_biases, (1, 0))
    return _tc_combine(acc, deg, x, bmw, bmb, bsw, bsb, linear_combinations)
